# baseline (speedup 1.0000x reference)
import functools

import jax
import jax.numpy as jnp
from jax import lax
from jax.experimental import pallas as pl
from jax.experimental.pallas import tpu as pltpu
from jax.experimental.pallas import tpu_sc as plsc

VOCAB = 1000000
FACT = 64
HIDDEN = 128
B = 16384
L = 50
N = B * L
NC, NS = 2, 16
NW = NC * NS
PER_W = N // NW
CHUNK = 128
NCHUNK = PER_W // CHUNK


def _sc_gather(table, idx):
    mesh = plsc.VectorSubcoreMesh(core_axis_name="c", subcore_axis_name="s")

    @functools.partial(
        pl.kernel,
        mesh=mesh,
        out_type=jax.ShapeDtypeStruct((N, FACT), jnp.float32),
        scratch_types=[
            pltpu.VMEM((PER_W,), jnp.int32),
            pltpu.VMEM((CHUNK, FACT), jnp.float32),
            pltpu.SemaphoreType.DMA,
        ],
        compiler_params=pltpu.CompilerParams(use_tc_tiling_on_sc=False),
    )
    def k(table_hbm, idx_hbm, emb_hbm, idx_v, rows_v, sem):
        wid = lax.axis_index("s") * NC + lax.axis_index("c")
        base = wid * PER_W
        pltpu.sync_copy(idx_hbm.at[pl.ds(base, PER_W)], idx_v)

        def body(c, carry):
            pltpu.async_copy(
                table_hbm.at[idx_v.at[pl.ds(c * CHUNK, CHUNK)]], rows_v, sem
            ).wait()
            pltpu.sync_copy(rows_v, emb_hbm.at[pl.ds(base + c * CHUNK, CHUNK)])
            return carry

        lax.fori_loop(0, NCHUNK, body, 0)

    return k(table, idx)


RB = 2048


def _proj_body(emb_ref, w_ref, out_ref):
    out_ref[...] = jnp.dot(
        emb_ref[...], w_ref[...], preferred_element_type=jnp.float32
    )


def _project(emb, W):
    return pl.pallas_call(
        _proj_body,
        grid=(N // RB,),
        in_specs=[
            pl.BlockSpec((RB, FACT), lambda i: (i, 0)),
            pl.BlockSpec((FACT, HIDDEN), lambda i: (0, 0)),
        ],
        out_specs=pl.BlockSpec((RB, HIDDEN), lambda i: (i, 0)),
        out_shape=jax.ShapeDtypeStruct((N, HIDDEN), jnp.float32),
    )(emb, W)


def kernel(inputs, table, W):
    idx = inputs.reshape(-1).astype(jnp.int32)
    emb = _sc_gather(table, idx)
    out = _project(emb, W)
    return out.reshape(B, L, HIDDEN)


# speedup vs baseline: 1.2733x; 1.2733x over previous
import functools

import jax
import jax.numpy as jnp
from jax import lax
from jax.experimental import pallas as pl
from jax.experimental.pallas import tpu as pltpu
from jax.experimental.pallas import tpu_sc as plsc

VOCAB = 1000000
FACT = 64
HIDDEN = 128
B = 16384
L = 50
N = B * L
NC, NS = 2, 16
NW = NC * NS
PER_W = N // NW
CHUNK = 128
NCHUNK = PER_W // CHUNK

VBLK = 8000


def _proj_body(t_ref, w_ref, out_ref):
    out_ref[...] = jnp.dot(
        t_ref[...], w_ref[...], preferred_element_type=jnp.float32
    )


def _project_table(table, W):
    return pl.pallas_call(
        _proj_body,
        grid=(VOCAB // VBLK,),
        in_specs=[
            pl.BlockSpec((VBLK, FACT), lambda i: (i, 0)),
            pl.BlockSpec((FACT, HIDDEN), lambda i: (0, 0)),
        ],
        out_specs=pl.BlockSpec((VBLK, HIDDEN), lambda i: (i, 0)),
        out_shape=jax.ShapeDtypeStruct((VOCAB, HIDDEN), jnp.float32),
    )(table, W)


def _sc_gather(tablew, idx):
    mesh = plsc.VectorSubcoreMesh(core_axis_name="c", subcore_axis_name="s")

    @functools.partial(
        pl.kernel,
        mesh=mesh,
        out_type=jax.ShapeDtypeStruct((N, HIDDEN), jnp.float32),
        scratch_types=[
            pltpu.VMEM((PER_W,), jnp.int32),
            pltpu.VMEM((CHUNK, HIDDEN), jnp.float32),
            pltpu.SemaphoreType.DMA,
        ],
    )
    def k(tablew_hbm, idx_hbm, out_hbm, idx_v, rows_v, sem):
        wid = lax.axis_index("s") * NC + lax.axis_index("c")
        base = wid * PER_W
        pltpu.sync_copy(idx_hbm.at[pl.ds(base, PER_W)], idx_v)

        def body(c, carry):
            pltpu.async_copy(
                tablew_hbm.at[idx_v.at[pl.ds(c * CHUNK, CHUNK)]], rows_v, sem
            ).wait()
            pltpu.sync_copy(rows_v, out_hbm.at[pl.ds(base + c * CHUNK, CHUNK)])
            return carry

        lax.fori_loop(0, NCHUNK, body, 0)

    return k(tablew, idx)


def kernel(inputs, table, W):
    idx = inputs.reshape(-1).astype(jnp.int32)
    tablew = _project_table(table, W)
    out = _sc_gather(tablew, idx)
    return out.reshape(B, L, HIDDEN)


# speedup vs baseline: 1.2735x; 1.0001x over previous
import functools

import jax
import jax.numpy as jnp
from jax import lax
from jax.experimental import pallas as pl
from jax.experimental.pallas import tpu as pltpu
from jax.experimental.pallas import tpu_sc as plsc

VOCAB = 1000000
FACT = 64
HIDDEN = 128
B = 16384
L = 50
N = B * L
NC, NS = 2, 16
NW = NC * NS
PER_W = N // NW
CHUNK = 128
NCHUNK = PER_W // CHUNK

VBLK = 8000


def _proj_body(t_ref, w_ref, out_ref):
    out_ref[...] = jnp.dot(
        t_ref[...], w_ref[...], preferred_element_type=jnp.float32
    )


def _project_table(table, W):
    return pl.pallas_call(
        _proj_body,
        grid=(VOCAB // VBLK,),
        in_specs=[
            pl.BlockSpec((VBLK, FACT), lambda i: (i, 0)),
            pl.BlockSpec((FACT, HIDDEN), lambda i: (0, 0)),
        ],
        out_specs=pl.BlockSpec((VBLK, HIDDEN), lambda i: (i, 0)),
        out_shape=jax.ShapeDtypeStruct((VOCAB, HIDDEN), jnp.float32),
    )(table, W)


def _sc_gather(tablew, idx):
    mesh = plsc.VectorSubcoreMesh(core_axis_name="c", subcore_axis_name="s")

    @functools.partial(
        pl.kernel,
        mesh=mesh,
        out_type=jax.ShapeDtypeStruct((N, HIDDEN), jnp.float32),
        scratch_types=[
            pltpu.VMEM((PER_W,), jnp.int32),
            pltpu.VMEM((CHUNK, HIDDEN), jnp.float32),
            pltpu.SemaphoreType.DMA,
        ],
        compiler_params=pltpu.CompilerParams(use_tc_tiling_on_sc=True),
    )
    def k(tablew_hbm, idx_hbm, out_hbm, idx_v, rows_v, sem):
        wid = lax.axis_index("s") * NC + lax.axis_index("c")
        base = wid * PER_W
        pltpu.sync_copy(idx_hbm.at[pl.ds(base, PER_W)], idx_v)

        def body(c, carry):
            pltpu.async_copy(
                tablew_hbm.at[idx_v.at[pl.ds(c * CHUNK, CHUNK)]], rows_v, sem
            ).wait()
            pltpu.sync_copy(rows_v, out_hbm.at[pl.ds(base + c * CHUNK, CHUNK)])
            return carry

        lax.fori_loop(0, NCHUNK, body, 0)

    return k(tablew, idx)


def kernel(inputs, table, W):
    idx = inputs.reshape(-1).astype(jnp.int32)
    tablew = _project_table(table, W)
    out = _sc_gather(tablew, idx)
    return out.reshape(B, L, HIDDEN)


# speedup vs baseline: 3.2957x; 2.5879x over previous
import functools

import jax
import jax.numpy as jnp
from jax import lax
from jax.experimental import pallas as pl
from jax.experimental.pallas import tpu as pltpu
from jax.experimental.pallas import tpu_sc as plsc

VOCAB = 1000000
FACT = 64
HIDDEN = 128
B = 16384
L = 50
N = B * L
NC, NS = 2, 16
NW = NC * NS
PER_W = N // NW
CHUNK = 128
NCHUNK = PER_W // CHUNK

VBLK = 8192


def _proj_body(tt_ref, w_ref, out_ref):
    out_ref[...] = lax.dot_general(
        tt_ref[...],
        w_ref[...],
        dimension_numbers=(((0,), (0,)), ((), ())),
        preferred_element_type=jnp.float32,
    )


def _project_table(table_t, W):
    return pl.pallas_call(
        _proj_body,
        grid=((VOCAB + VBLK - 1) // VBLK,),
        in_specs=[
            pl.BlockSpec((FACT, VBLK), lambda i: (0, i)),
            pl.BlockSpec((FACT, HIDDEN), lambda i: (0, 0)),
        ],
        out_specs=pl.BlockSpec((VBLK, HIDDEN), lambda i: (i, 0)),
        out_shape=jax.ShapeDtypeStruct((VOCAB, HIDDEN), jnp.float32),
    )(table_t, W)


def _sc_gather(tablew, idx):
    mesh = plsc.VectorSubcoreMesh(core_axis_name="c", subcore_axis_name="s")

    @functools.partial(
        pl.kernel,
        mesh=mesh,
        out_type=jax.ShapeDtypeStruct((N, HIDDEN), jnp.float32),
        scratch_types=[
            pltpu.VMEM((PER_W,), jnp.int32),
            pltpu.VMEM((CHUNK, HIDDEN), jnp.float32),
            pltpu.SemaphoreType.DMA,
        ],
        compiler_params=pltpu.CompilerParams(use_tc_tiling_on_sc=True),
    )
    def k(tablew_hbm, idx_hbm, out_hbm, idx_v, rows_v, sem):
        wid = lax.axis_index("s") * NC + lax.axis_index("c")
        base = wid * PER_W
        pltpu.sync_copy(idx_hbm.at[pl.ds(base, PER_W)], idx_v)

        def body(c, carry):
            pltpu.async_copy(
                tablew_hbm.at[idx_v.at[pl.ds(c * CHUNK, CHUNK)]], rows_v, sem
            ).wait()
            pltpu.sync_copy(rows_v, out_hbm.at[pl.ds(base + c * CHUNK, CHUNK)])
            return carry

        lax.fori_loop(0, NCHUNK, body, 0)

    return k(tablew, idx)


def kernel(inputs, table, W):
    idx = inputs.astype(jnp.int32).T.reshape(-1)
    tablew = _project_table(table.T, W)
    out = _sc_gather(tablew, idx)
    return out.reshape(L, B, HIDDEN).transpose(1, 0, 2)


# speedup vs baseline: 3.9638x; 1.2027x over previous
import functools

import jax
import jax.numpy as jnp
from jax import lax
from jax.experimental import pallas as pl
from jax.experimental.pallas import tpu as pltpu
from jax.experimental.pallas import tpu_sc as plsc

VOCAB = 1000000
FACT = 64
HIDDEN = 128
B = 16384
L = 50
N = B * L
NC, NS = 2, 16
NW = NC * NS
PER_W = N // NW
CHUNK = 128
NCHUNK = PER_W // CHUNK
NBUF = 4
NGRP = NCHUNK // NBUF

VBLK = 8192


def _proj_body(tt_ref, w_ref, out_ref):
    out_ref[...] = lax.dot_general(
        tt_ref[...],
        w_ref[...],
        dimension_numbers=(((0,), (0,)), ((), ())),
        preferred_element_type=jnp.float32,
    )


def _project_table(table_t, W):
    return pl.pallas_call(
        _proj_body,
        grid=((VOCAB + VBLK - 1) // VBLK,),
        in_specs=[
            pl.BlockSpec((FACT, VBLK), lambda i: (0, i)),
            pl.BlockSpec((FACT, HIDDEN), lambda i: (0, 0)),
        ],
        out_specs=pl.BlockSpec((VBLK, HIDDEN), lambda i: (i, 0)),
        out_shape=jax.ShapeDtypeStruct((VOCAB, HIDDEN), jnp.float32),
    )(table_t, W)


def _sc_gather(tablew, idx):
    mesh = plsc.VectorSubcoreMesh(core_axis_name="c", subcore_axis_name="s")

    @functools.partial(
        pl.kernel,
        mesh=mesh,
        out_type=jax.ShapeDtypeStruct((N, HIDDEN), jnp.float32),
        scratch_types=[
            pltpu.VMEM((PER_W,), jnp.int32),
            pltpu.VMEM((NBUF, CHUNK, HIDDEN), jnp.float32),
            pltpu.SemaphoreType.DMA,
            pltpu.SemaphoreType.DMA,
        ],
        compiler_params=pltpu.CompilerParams(use_tc_tiling_on_sc=True),
    )
    def k(tablew_hbm, idx_hbm, out_hbm, idx_v, rows_v, gsem, ssem):
        wid = lax.axis_index("s") * NC + lax.axis_index("c")
        base = wid * PER_W
        pltpu.sync_copy(idx_hbm.at[pl.ds(base, PER_W)], idx_v)

        def gather(c, b):
            return pltpu.make_async_copy(
                tablew_hbm.at[idx_v.at[pl.ds(c * CHUNK, CHUNK)]],
                rows_v.at[b],
                gsem,
            )

        def store(c, b):
            return pltpu.make_async_copy(
                rows_v.at[b],
                out_hbm.at[pl.ds(base + c * CHUNK, CHUNK)],
                ssem,
            )

        def body(h, carry):
            c0 = h * NBUF
            for b in range(NBUF):
                gather(c0 + b, b).start()
            for b in range(NBUF):
                gather(c0 + b, b).wait()
            for b in range(NBUF):
                store(c0 + b, b).start()
            for b in range(NBUF):
                store(c0 + b, b).wait()
            return carry

        lax.fori_loop(0, NGRP, body, 0)

    return k(tablew, idx)


def kernel(inputs, table, W):
    idx = inputs.astype(jnp.int32).T.reshape(-1)
    tablew = _project_table(table.T, W)
    out = _sc_gather(tablew, idx)
    return out.reshape(L, B, HIDDEN).transpose(1, 0, 2)


# speedup vs baseline: 4.0318x; 1.0172x over previous
import functools

import jax
import jax.numpy as jnp
from jax import lax
from jax.experimental import pallas as pl
from jax.experimental.pallas import tpu as pltpu
from jax.experimental.pallas import tpu_sc as plsc

VOCAB = 1000000
FACT = 64
HIDDEN = 128
B = 16384
L = 50
N = B * L
NC, NS = 2, 16
NW = NC * NS
PER_W = N // NW
CHUNK = 128
NCHUNK = PER_W // CHUNK
NBUF = 4
NGRP = NCHUNK // NBUF

VBLK = 8192


def _proj_body(tt_ref, w_ref, out_ref):
    out_ref[...] = lax.dot_general(
        tt_ref[...],
        w_ref[...],
        dimension_numbers=(((0,), (0,)), ((), ())),
        preferred_element_type=jnp.float32,
    )


def _project_table(table_t, W):
    return pl.pallas_call(
        _proj_body,
        grid=((VOCAB + VBLK - 1) // VBLK,),
        in_specs=[
            pl.BlockSpec((FACT, VBLK), lambda i: (0, i)),
            pl.BlockSpec((FACT, HIDDEN), lambda i: (0, 0)),
        ],
        out_specs=pl.BlockSpec((VBLK, HIDDEN), lambda i: (i, 0)),
        out_shape=jax.ShapeDtypeStruct((VOCAB, HIDDEN), jnp.float32),
    )(table_t, W)


def _sc_gather(tablew, idx):
    mesh = plsc.VectorSubcoreMesh(core_axis_name="c", subcore_axis_name="s")

    @functools.partial(
        pl.kernel,
        mesh=mesh,
        out_type=jax.ShapeDtypeStruct((N, HIDDEN), jnp.float32),
        scratch_types=[
            pltpu.VMEM((PER_W,), jnp.int32),
            pltpu.VMEM((NBUF, CHUNK, HIDDEN), jnp.float32),
            pltpu.SemaphoreType.DMA,
            pltpu.SemaphoreType.DMA,
            pltpu.SemaphoreType.DMA,
        ],
        compiler_params=pltpu.CompilerParams(use_tc_tiling_on_sc=True),
    )
    def k(tablew_hbm, idx_hbm, out_hbm, idx_v, rows_v, gsem_a, gsem_b, ssem):
        wid = lax.axis_index("s") * NC + lax.axis_index("c")
        base = wid * PER_W
        pltpu.sync_copy(idx_hbm.at[pl.ds(base, PER_W)], idx_v)

        def gather(c, b, sem):
            return pltpu.make_async_copy(
                tablew_hbm.at[idx_v.at[pl.ds(c * CHUNK, CHUNK)]],
                rows_v.at[b],
                sem,
            )

        def store(c, b):
            return pltpu.make_async_copy(
                rows_v.at[b],
                out_hbm.at[pl.ds(base + c * CHUNK, CHUNK)],
                ssem,
            )

        half = NBUF // 2

        def body(h, carry):
            c0 = h * NBUF
            for b in range(half):
                gather(c0 + b, b, gsem_a).start()
            for b in range(half, NBUF):
                gather(c0 + b, b, gsem_b).start()
            for b in range(half):
                gather(c0 + b, b, gsem_a).wait()
            for b in range(half):
                store(c0 + b, b).start()
            for b in range(half, NBUF):
                gather(c0 + b, b, gsem_b).wait()
            for b in range(half, NBUF):
                store(c0 + b, b).start()
            for b in range(NBUF):
                store(c0 + b, b).wait()
            return carry

        lax.fori_loop(0, NGRP, body, 0)

    return k(tablew, idx)


def kernel(inputs, table, W):
    idx = inputs.astype(jnp.int32).T.reshape(-1)
    tablew = _project_table(table.T, W)
    out = _sc_gather(tablew, idx)
    return out.reshape(L, B, HIDDEN).transpose(1, 0, 2)


# speedup vs baseline: 4.2244x; 1.0478x over previous
import functools

import jax
import jax.numpy as jnp
from jax import lax
from jax.experimental import pallas as pl
from jax.experimental.pallas import tpu as pltpu
from jax.experimental.pallas import tpu_sc as plsc

VOCAB = 1000000
FACT = 64
HIDDEN = 128
B = 16384
L = 50
N = B * L
NC, NS = 2, 16
NW = NC * NS
PER_W = N // NW
CHUNK = 128
NCHUNK = PER_W // CHUNK
NBUF = 4
NGRP = NCHUNK // NBUF

VBLK = 16384


def _proj_body(tt_ref, w_ref, out_ref):
    out_ref[...] = lax.dot_general(
        tt_ref[...],
        w_ref[...],
        dimension_numbers=(((0,), (0,)), ((), ())),
        preferred_element_type=jnp.float32,
    )


def _project_table(table_t, W):
    return pl.pallas_call(
        _proj_body,
        grid=((VOCAB + VBLK - 1) // VBLK,),
        in_specs=[
            pl.BlockSpec((FACT, VBLK), lambda i: (0, i)),
            pl.BlockSpec((FACT, HIDDEN), lambda i: (0, 0)),
        ],
        out_specs=pl.BlockSpec((VBLK, HIDDEN), lambda i: (i, 0)),
        out_shape=jax.ShapeDtypeStruct((VOCAB, HIDDEN), jnp.float32),
    )(table_t, W)


def _sc_gather(tablew, idx):
    mesh = plsc.VectorSubcoreMesh(core_axis_name="c", subcore_axis_name="s")

    @functools.partial(
        pl.kernel,
        mesh=mesh,
        out_type=jax.ShapeDtypeStruct((N, HIDDEN), jnp.float32),
        scratch_types=[
            pltpu.VMEM((PER_W,), jnp.int32),
            pltpu.VMEM((NBUF, CHUNK, HIDDEN), jnp.float32),
            pltpu.SemaphoreType.DMA,
            pltpu.SemaphoreType.DMA,
            pltpu.SemaphoreType.DMA,
        ],
        compiler_params=pltpu.CompilerParams(use_tc_tiling_on_sc=True),
    )
    def k(tablew_hbm, idx_hbm, out_hbm, idx_v, rows_v, gsem_a, gsem_b, ssem):
        wid = lax.axis_index("s") * NC + lax.axis_index("c")
        base = wid * PER_W
        pltpu.sync_copy(idx_hbm.at[pl.ds(base, PER_W)], idx_v)

        def gather(c, b, sem):
            return pltpu.make_async_copy(
                tablew_hbm.at[idx_v.at[pl.ds(c * CHUNK, CHUNK)]],
                rows_v.at[b],
                sem,
            )

        def store(c, b):
            return pltpu.make_async_copy(
                rows_v.at[b],
                out_hbm.at[pl.ds(base + c * CHUNK, CHUNK)],
                ssem,
            )

        half = NBUF // 2

        def body(h, carry):
            c0 = h * NBUF
            for b in range(half):
                gather(c0 + b, b, gsem_a).start()
            for b in range(half, NBUF):
                gather(c0 + b, b, gsem_b).start()
            for b in range(half):
                gather(c0 + b, b, gsem_a).wait()
            for b in range(half):
                store(c0 + b, b).start()
            for b in range(half, NBUF):
                gather(c0 + b, b, gsem_b).wait()
            for b in range(half, NBUF):
                store(c0 + b, b).start()
            for b in range(NBUF):
                store(c0 + b, b).wait()
            return carry

        lax.fori_loop(0, NGRP, body, 0)

    return k(tablew, idx)


def kernel(inputs, table, W):
    idx = inputs.astype(jnp.int32).T.reshape(-1)
    tablew = _project_table(table.T, W)
    out = _sc_gather(tablew, idx)
    return out.reshape(L, B, HIDDEN).transpose(1, 0, 2)


# speedup vs baseline: 4.2655x; 1.0097x over previous
import functools

import jax
import jax.numpy as jnp
from jax import lax
from jax.experimental import pallas as pl
from jax.experimental.pallas import tpu as pltpu
from jax.experimental.pallas import tpu_sc as plsc

VOCAB = 1000000
FACT = 64
HIDDEN = 128
B = 16384
L = 50
N = B * L
NC, NS = 2, 16
NW = NC * NS
PER_W = N // NW
CHUNK = 128
NCHUNK = PER_W // CHUNK
NBUF = 4
NGRP = NCHUNK // NBUF

VBLK = 32768


def _proj_body(tt_ref, w_ref, out_ref):
    out_ref[...] = lax.dot_general(
        tt_ref[...],
        w_ref[...],
        dimension_numbers=(((0,), (0,)), ((), ())),
        preferred_element_type=jnp.float32,
    )


def _project_table(table_t, W):
    return pl.pallas_call(
        _proj_body,
        grid=((VOCAB + VBLK - 1) // VBLK,),
        in_specs=[
            pl.BlockSpec((FACT, VBLK), lambda i: (0, i)),
            pl.BlockSpec((FACT, HIDDEN), lambda i: (0, 0)),
        ],
        out_specs=pl.BlockSpec((VBLK, HIDDEN), lambda i: (i, 0)),
        out_shape=jax.ShapeDtypeStruct((VOCAB, HIDDEN), jnp.float32),
    )(table_t, W)


def _sc_gather(tablew, idx):
    mesh = plsc.VectorSubcoreMesh(core_axis_name="c", subcore_axis_name="s")

    @functools.partial(
        pl.kernel,
        mesh=mesh,
        out_type=jax.ShapeDtypeStruct((N, HIDDEN), jnp.float32),
        scratch_types=[
            pltpu.VMEM((PER_W,), jnp.int32),
            pltpu.VMEM((NBUF, CHUNK, HIDDEN), jnp.float32),
            pltpu.SemaphoreType.DMA,
            pltpu.SemaphoreType.DMA,
            pltpu.SemaphoreType.DMA,
        ],
        compiler_params=pltpu.CompilerParams(use_tc_tiling_on_sc=True),
    )
    def k(tablew_hbm, idx_hbm, out_hbm, idx_v, rows_v, gsem_a, gsem_b, ssem):
        wid = lax.axis_index("s") * NC + lax.axis_index("c")
        base = wid * PER_W
        pltpu.sync_copy(idx_hbm.at[pl.ds(base, PER_W)], idx_v)

        def gather(c, b, sem):
            return pltpu.make_async_copy(
                tablew_hbm.at[idx_v.at[pl.ds(c * CHUNK, CHUNK)]],
                rows_v.at[b],
                sem,
            )

        def store(c, b):
            return pltpu.make_async_copy(
                rows_v.at[b],
                out_hbm.at[pl.ds(base + c * CHUNK, CHUNK)],
                ssem,
            )

        half = NBUF // 2

        def body(h, carry):
            c0 = h * NBUF
            for b in range(half):
                gather(c0 + b, b, gsem_a).start()
            for b in range(half, NBUF):
                gather(c0 + b, b, gsem_b).start()
            for b in range(half):
                gather(c0 + b, b, gsem_a).wait()
            for b in range(half):
                store(c0 + b, b).start()
            for b in range(half, NBUF):
                gather(c0 + b, b, gsem_b).wait()
            for b in range(half, NBUF):
                store(c0 + b, b).start()
            for b in range(NBUF):
                store(c0 + b, b).wait()
            return carry

        lax.fori_loop(0, NGRP, body, 0)

    return k(tablew, idx)


def kernel(inputs, table, W):
    idx = inputs.astype(jnp.int32).T.reshape(-1)
    tablew = _project_table(table.T, W)
    out = _sc_gather(tablew, idx)
    return out.reshape(L, B, HIDDEN).transpose(1, 0, 2)


# speedup vs baseline: 4.2674x; 1.0005x over previous
import functools

import jax
import jax.numpy as jnp
from jax import lax
from jax.experimental import pallas as pl
from jax.experimental.pallas import tpu as pltpu
from jax.experimental.pallas import tpu_sc as plsc

VOCAB = 1000000
FACT = 64
HIDDEN = 128
B = 16384
L = 50
N = B * L
NC, NS = 2, 16
NW = NC * NS
PER_W = N // NW
CHUNK = 128
NCHUNK = PER_W // CHUNK
NBUF = 4
NGRP = NCHUNK // NBUF

VBLK = 32768


def _proj_body(tt_ref, w_ref, out_ref):
    out_ref[...] = lax.dot_general(
        tt_ref[...],
        w_ref[...],
        dimension_numbers=(((0,), (0,)), ((), ())),
        preferred_element_type=jnp.float32,
    )


def _project_table(table_t, W):
    return pl.pallas_call(
        _proj_body,
        grid=((VOCAB + VBLK - 1) // VBLK,),
        in_specs=[
            pl.BlockSpec((FACT, VBLK), lambda i: (0, i)),
            pl.BlockSpec((FACT, HIDDEN), lambda i: (0, 0)),
        ],
        out_specs=pl.BlockSpec((VBLK, HIDDEN), lambda i: (i, 0)),
        out_shape=jax.ShapeDtypeStruct((VOCAB, HIDDEN), jnp.float32),
    )(table_t, W)


def _sc_gather(tablew, idx):
    mesh = plsc.VectorSubcoreMesh(core_axis_name="c", subcore_axis_name="s")

    @functools.partial(
        pl.kernel,
        mesh=mesh,
        out_type=jax.ShapeDtypeStruct((N, HIDDEN), jnp.float32),
        scratch_types=[
            pltpu.VMEM((PER_W,), jnp.int32),
            pltpu.VMEM((2, 2 * CHUNK, HIDDEN), jnp.float32),
            pltpu.SemaphoreType.DMA,
            pltpu.SemaphoreType.DMA,
            pltpu.SemaphoreType.DMA,
        ],
        compiler_params=pltpu.CompilerParams(use_tc_tiling_on_sc=True),
    )
    def k(tablew_hbm, idx_hbm, out_hbm, idx_v, rows_v, gsem_a, gsem_b, ssem):
        wid = lax.axis_index("s") * NC + lax.axis_index("c")
        base = wid * PER_W
        pltpu.sync_copy(idx_hbm.at[pl.ds(base, PER_W)], idx_v)

        def gather(c, b, j, sem):
            return pltpu.make_async_copy(
                tablew_hbm.at[idx_v.at[pl.ds(c * CHUNK, CHUNK)]],
                rows_v.at[b, pl.ds(j * CHUNK, CHUNK)],
                sem,
            )

        def store(c, b):
            return pltpu.make_async_copy(
                rows_v.at[b],
                out_hbm.at[pl.ds(base + c * CHUNK, 2 * CHUNK)],
                ssem,
            )

        def body(h, carry):
            c0 = h * NBUF
            for j in range(2):
                gather(c0 + j, 0, j, gsem_a).start()
            for j in range(2):
                gather(c0 + 2 + j, 1, j, gsem_b).start()
            for j in range(2):
                gather(c0 + j, 0, j, gsem_a).wait()
            store(c0, 0).start()
            for j in range(2):
                gather(c0 + 2 + j, 1, j, gsem_b).wait()
            store(c0 + 2, 1).start()
            store(c0, 0).wait()
            store(c0 + 2, 1).wait()
            return carry

        lax.fori_loop(0, NGRP, body, 0)

    return k(tablew, idx)


def kernel(inputs, table, W):
    idx = inputs.astype(jnp.int32).T.reshape(-1)
    tablew = _project_table(table.T, W)
    out = _sc_gather(tablew, idx)
    return out.reshape(L, B, HIDDEN).transpose(1, 0, 2)
